# Initial kernel scaffold; baseline (speedup 1.0000x reference)
import jax
import jax.numpy as jnp
from jax.experimental import pallas as pl


def kernel(soft_weight, gt_boxes_batch_ids, gt_boxes_count):
    raise NotImplementedError("write your pallas kernel here")


# speedup vs baseline: 8.0411x; 8.0411x over previous
import functools

import numpy as np

import jax
import jax.numpy as jnp
from jax import lax
from jax.experimental import pallas as pl
from jax.experimental.pallas import tpu as pltpu
from jax.experimental.pallas import tpu_sc as plsc

B = 4
N = 16384
F = 128
MAX_GT = 8192
TOP_K = 3

L = 16
KV = F // L
CH = 128
NW = 32
CHUNKS_PER_BATCH = MAX_GT // CH
TOTAL_CHUNKS = B * CHUNKS_PER_BATCH
STEPS = TOTAL_CHUNKS // NW

_NEG = np.float32(-3.4028234663852886e38)
_IMIN = np.int32(-2147483648)


def _sc_body(soft_hbm, params_hbm, out_hbm, pvec, vin, vout, negbuf):
    wid = lax.axis_index("s") * 2 + lax.axis_index("c")
    pltpu.sync_copy(params_hbm, pvec)
    lanes = lax.iota(jnp.int32, L)
    pv = pvec[...]

    def extract(idx):
        return jnp.max(jnp.where(lanes == idx, pv, _IMIN))

    negv = jnp.full((L,), _NEG)
    none = jnp.full((L,), jnp.float32(-1.0))

    def fill_neg(r, _):
        for k in range(KV):
            negbuf[r, pl.ds(L * k, L)] = none
        return 0

    lax.fori_loop(0, CH, fill_neg, 0)

    def row_body(dshift, r, _):
        rr = r + dshift
        x = [vin[rr, pl.ds(L * k, L)] for k in range(KV)]
        a1 = x[0]
        a2 = negv
        a3 = negv
        for k in range(1, KV):
            t1 = jnp.maximum(a1, x[k])
            c2 = jnp.minimum(a1, x[k])
            t2 = jnp.maximum(a2, c2)
            c3 = jnp.minimum(a2, c2)
            a3 = jnp.maximum(a3, c3)
            a1 = t1
            a2 = t2
        m1 = jnp.max(a1)
        m1b = jnp.full((L,), m1)
        e1 = (jnp.where(a1 == m1b, 1, 0) + jnp.where(a2 == m1b, 1, 0)
              + jnp.where(a3 == m1b, 1, 0))
        n1 = jnp.sum(e1)
        b1 = jnp.where(a1 < m1b, a1, negv)
        b2 = jnp.where(a2 < m1b, a2, negv)
        b3 = jnp.where(a3 < m1b, a3, negv)
        m2 = jnp.max(jnp.maximum(jnp.maximum(b1, b2), b3))
        m2b = jnp.full((L,), m2)
        e2 = (jnp.where(b1 == m2b, 1, 0) + jnp.where(b2 == m2b, 1, 0)
              + jnp.where(b3 == m2b, 1, 0))
        n2 = jnp.sum(e2)
        d1 = jnp.where(b1 < m2b, b1, negv)
        d2 = jnp.where(b2 < m2b, b2, negv)
        d3 = jnp.where(b3 < m2b, b3, negv)
        m3 = jnp.max(jnp.maximum(jnp.maximum(d1, d2), d3))
        thr = jnp.where(n1 >= TOP_K, m1, jnp.where(n1 + n2 >= TOP_K, m2, m3))
        thrb = jnp.full((L,), thr)
        zero = jnp.zeros((L,), jnp.float32)
        for k in range(KV):
            vout[r, pl.ds(L * k, L)] = jnp.where(x[k] >= thrb, x[k], zero)
        return 0

    for t in range(STEPS):
        if t % 2 == 0:
            cg = jnp.int32(t * NW) + wid
        else:
            cg = jnp.int32(t * NW + NW - 1) - wid
        bi = cg // CHUNKS_PER_BATCH
        cl = cg % CHUNKS_PER_BATCH
        start = extract(bi)
        cnt = extract(bi + B)
        src = start + cl * CH
        vc = jnp.clip(jnp.minimum(cnt, MAX_GT) - cl * CH, 0, CH)
        srcc = jnp.minimum(src, N - CH)
        dshift = src - srcc
        out_at = out_hbm.at[pl.ds(cg * CH, CH)]

        @pl.when(vc > 0)
        def _():
            pltpu.sync_copy(soft_hbm.at[pl.ds(srcc, CH)], vin)
            lax.fori_loop(0, vc, functools.partial(row_body, dshift), 0)

            def fill_row(r, _):
                for k in range(KV):
                    vout[r, pl.ds(L * k, L)] = none
                return 0

            lax.fori_loop(vc, CH, fill_row, 0)
            pltpu.sync_copy(vout, out_at)

        @pl.when(vc <= 0)
        def _():
            pltpu.sync_copy(negbuf, out_at)


@jax.jit
def kernel(soft_weight, gt_boxes_batch_ids, gt_boxes_count):
    del gt_boxes_batch_ids
    counts = gt_boxes_count[:, 0].astype(jnp.int32)
    starts = jnp.concatenate(
        [jnp.zeros((1,), jnp.int32), jnp.cumsum(counts)[:-1].astype(jnp.int32)]
    )
    params = jnp.concatenate(
        [starts, counts, jnp.zeros((L - 2 * B,), jnp.int32)]
    )

    mesh = plsc.VectorSubcoreMesh(core_axis_name="c", subcore_axis_name="s")
    out = pl.kernel(
        _sc_body,
        out_type=jax.ShapeDtypeStruct((B * MAX_GT, F), jnp.float32),
        mesh=mesh,
        compiler_params=pltpu.CompilerParams(
            use_tc_tiling_on_sc=False, needs_layout_passes=False
        ),
        scratch_types=[
            pltpu.VMEM((L,), jnp.int32),
            pltpu.VMEM((CH, F), jnp.float32),
            pltpu.VMEM((CH, F), jnp.float32),
            pltpu.VMEM((CH, F), jnp.float32),
        ],
    )(soft_weight, params)
    return out.reshape(B, MAX_GT, F)
